# Initial kernel scaffold; baseline (speedup 1.0000x reference)
import jax
import jax.numpy as jnp
from jax.experimental import pallas as pl


def kernel(header_scalar, called_ids, hand_ids, blind_ids, bury_ids, trick_card_ids, trick_is_picker, trick_is_partner_known, card_table, seat_table, role_table, W_trick, b_trick, W_simple, b_simple, q_hand, Wk_hand, Wv_hand, q_trick, Wk_trick, Wv_trick, q_blind, Wk_blind, Wv_blind, q_bury, Wk_bury, Wv_bury, W_head, b_head):
    raise NotImplementedError("write your pallas kernel here")


# speedup vs baseline: 9.0275x; 9.0275x over previous
import functools
import math

import jax
import jax.numpy as jnp
from jax.experimental import pallas as pl


def _silu(x):
    return x * jax.nn.sigmoid(x)


def _body(hs_ref, called_ref, hand_ref, blind_ref, bury_ref,
          tcard_ref, tpick_ref, tpart_ref,
          card_t_ref, seat_t_ref, role_t_ref,
          W_trick_ref, b_trick_ref, W_simple_ref, b_simple_ref,
          q_hand_ref, Wk_hand_ref, Wv_hand_ref,
          q_trick_ref, Wk_trick_ref, Wv_trick_ref,
          q_blind_ref, Wk_blind_ref, Wv_blind_ref,
          q_bury_ref, Wk_bury_ref, Wv_bury_ref,
          W_head_ref, b_head_ref, out_ref):
    f32 = jnp.float32
    card_table = card_t_ref[...]
    seat_table = seat_t_ref[...]
    role_table = role_t_ref[...]
    W_simple = W_simple_ref[...]
    b_simple = b_simple_ref[...]
    W_trick = W_trick_ref[...]
    b_trick = b_trick_ref[...]

    def matT(a, b):
        return jax.lax.dot_general(a, b, (((1,), (1,)), ((), ())),
                                   preferred_element_type=f32)

    st = _silu(matT(card_table, W_simple) + b_simple)
    inv_sqrt = 1.0 / math.sqrt(32.0)

    def score_v(tok, q_ref, Wk_ref, Wv_ref):
        qW = jnp.dot(q_ref[...], Wk_ref[...],
                     preferred_element_type=f32)
        score = matT(tok, qW) * inv_sqrt
        v = matT(tok, Wv_ref[...])
        return score, v

    score_h, v_h = score_v(st, q_hand_ref, Wk_hand_ref, Wv_hand_ref)
    score_b, v_b = score_v(st, q_blind_ref, Wk_blind_ref, Wv_blind_ref)
    score_y, v_y = score_v(st, q_bury_ref, Wk_bury_ref, Wv_bury_ref)

    ridx = jax.lax.broadcasted_iota(jnp.int32, (680, 1), 0)
    c_id = ridx % 34
    r_id = (ridx // 34) % 4
    p_id = ridx // 136
    ohc = (c_id == jax.lax.broadcasted_iota(jnp.int32, (1, 34), 1)).astype(f32)
    ohs = ((p_id + 1) == jax.lax.broadcasted_iota(jnp.int32, (1, 6), 1)).astype(f32)
    ohr = (r_id == jax.lax.broadcasted_iota(jnp.int32, (1, 4), 1)).astype(f32)
    Xc = jnp.dot(ohc, card_table, preferred_element_type=f32)
    Xs = jnp.dot(ohs, seat_table, preferred_element_type=f32)
    Xr = jnp.dot(ohr, role_table, preferred_element_type=f32)
    X = jnp.concatenate([Xc, Xs, Xr], axis=1)
    tt = _silu(matT(X, W_trick) + b_trick)
    score_t, v_t = score_v(tt, q_trick_ref, Wk_trick_ref, Wv_trick_ref)

    def pool(ids, sub_idx, T, score_tab, v_tab):
        P = ids.shape[1]
        iota_t = jax.lax.broadcasted_iota(jnp.int32, (1, T), 1)
        cols = []
        for j in range(P):
            oh = (sub_idx[:, j:j + 1] == iota_t).astype(f32)
            cols.append(jnp.dot(oh, score_tab, preferred_element_type=f32))
        s = jnp.concatenate(cols, axis=1)
        mask = ids != 0
        att = jnp.where(mask, s, -1000000000.0)
        m = jnp.max(att, axis=1, keepdims=True)
        e = jnp.exp(att - m)
        z = jnp.sum(e, axis=1, keepdims=True)
        valid = jnp.any(mask, axis=1, keepdims=True)
        w = jnp.where(valid, e / z, 0.0)
        acc = None
        for j in range(P):
            oh = (sub_idx[:, j:j + 1] == iota_t).astype(f32)
            a = w[:, j:j + 1] * oh
            part = jnp.dot(a, v_tab, preferred_element_type=f32)
            acc = part if acc is None else acc + part
        return acc

    hand_ids = hand_ref[...]
    blind_ids = blind_ref[...]
    bury_ids = bury_ref[...]
    tcard = tcard_ref[...]
    role = tpick_ref[...] + 2 * tpart_ref[...]

    out_h = pool(hand_ids, hand_ids, 34, score_h, v_h)
    out_b = pool(blind_ids, blind_ids, 34, score_b, v_b)
    out_y = pool(bury_ids, bury_ids, 34, score_y, v_y)

    sub = role * 34 + tcard
    iota136 = jax.lax.broadcasted_iota(jnp.int32, (1, 136), 1)
    cols = []
    for j in range(5):
        oh = (sub[:, j:j + 1] == iota136).astype(f32)
        cols.append(jnp.dot(oh, score_t[j * 136:(j + 1) * 136],
                            preferred_element_type=f32))
    s = jnp.concatenate(cols, axis=1)
    mask = tcard != 0
    att = jnp.where(mask, s, -1000000000.0)
    m = jnp.max(att, axis=1, keepdims=True)
    e = jnp.exp(att - m)
    z = jnp.sum(e, axis=1, keepdims=True)
    valid = jnp.any(mask, axis=1, keepdims=True)
    w = jnp.where(valid, e / z, 0.0)
    out_t = None
    for j in range(5):
        oh = (sub[:, j:j + 1] == iota136).astype(f32)
        a = w[:, j:j + 1] * oh
        part = jnp.dot(a, v_t[j * 136:(j + 1) * 136],
                       preferred_element_type=f32)
        out_t = part if out_t is None else out_t + part

    hcol = jax.lax.broadcasted_iota(jnp.int32, (1, 10), 1)
    inv_norm = jnp.where(hcol == 3, 1.0 / 6.0,
                         jnp.where(hcol >= 6, 0.2, 1.0)).astype(f32)
    hs = hs_ref[...] * inv_norm
    W_head = W_head_ref[...]
    ct_head = matT(card_table, W_head[:, 10:18])
    oh_called = (called_ref[...] ==
                 jax.lax.broadcasted_iota(jnp.int32, (1, 34), 1)).astype(f32)
    header = _silu(matT(hs, W_head[:, 0:10]) +
                   jnp.dot(oh_called, ct_head, preferred_element_type=f32) +
                   b_head_ref[...])

    out_ref[...] = jnp.concatenate([out_h, out_t, out_b, out_y, header],
                                   axis=1)


def kernel(header_scalar, called_ids, hand_ids, blind_ids, bury_ids,
           trick_card_ids, trick_is_picker, trick_is_partner_known,
           card_table, seat_table, role_table,
           W_trick, b_trick, W_simple, b_simple,
           q_hand, Wk_hand, Wv_hand, q_trick, Wk_trick, Wv_trick,
           q_blind, Wk_blind, Wv_blind, q_bury, Wk_bury, Wv_bury,
           W_head, b_head):
    B = header_scalar.shape[0]
    BB = 2048
    grid = (B // BB,)
    f32 = jnp.float32

    def row_spec(width):
        return pl.BlockSpec((BB, width), lambda i: (i, 0))

    def full_spec(shape):
        return pl.BlockSpec(shape, lambda i: tuple(0 for _ in shape))

    called2 = called_ids.reshape(B, 1).astype(jnp.int32)
    row2 = lambda x: x.astype(jnp.int32)

    args = (header_scalar, called2,
            row2(hand_ids), row2(blind_ids), row2(bury_ids),
            row2(trick_card_ids), row2(trick_is_picker),
            row2(trick_is_partner_known),
            card_table, seat_table, role_table,
            W_trick, b_trick.reshape(1, -1), W_simple, b_simple.reshape(1, -1),
            q_hand.reshape(1, -1), Wk_hand, Wv_hand,
            q_trick.reshape(1, -1), Wk_trick, Wv_trick,
            q_blind.reshape(1, -1), Wk_blind, Wv_blind,
            q_bury.reshape(1, -1), Wk_bury, Wv_bury,
            W_head, b_head.reshape(1, -1))

    in_specs = [row_spec(10), row_spec(1),
                row_spec(8), row_spec(2), row_spec(2),
                row_spec(5), row_spec(5), row_spec(5)]
    in_specs += [full_spec(a.shape) for a in args[8:]]

    return pl.pallas_call(
        _body,
        grid=grid,
        in_specs=in_specs,
        out_specs=pl.BlockSpec((BB, 256), lambda i: (i, 0)),
        out_shape=jax.ShapeDtypeStruct((B, 256), f32),
    )(*args)
